# baseline (speedup 1.0000x reference)
import jax
import jax.numpy as jnp
from jax.experimental import pallas as pl
from jax.experimental.pallas import tpu as pltpu


def _masked_copy_body(x_ref, mid_ref, o_ref):
    o_ref[...] = x_ref[...]


def kernel(fea, mask_id):
    b, c, h, w = fea.shape
    hw = h * w
    x = fea.reshape(b, c, hw)
    cb = 128
    out = pl.pallas_call(
        _masked_copy_body,
        grid=(b, c // cb),
        in_specs=[
            pl.BlockSpec((1, cb, hw), lambda i, j: (i, j, 0)),
            pl.BlockSpec(memory_space=pltpu.SMEM),
        ],
        out_specs=pl.BlockSpec((1, cb, hw), lambda i, j: (i, j, 0)),
        out_shape=jax.ShapeDtypeStruct((b, c, hw), jnp.float32),
    )(x, mask_id)
    return out.reshape(b, c, h, w)


# speedup vs baseline: 1.3197x; 1.3197x over previous
import jax
import jax.numpy as jnp
from jax.experimental import pallas as pl
from jax.experimental.pallas import tpu as pltpu

_K = 4


def _body(x_ref, mid_ref, o_ref, ibuf, obuf, isem, osem):
    n = pl.num_programs(0)
    i = pl.program_id(0)
    slot = jax.lax.rem(i, _K)
    hw = x_ref.shape[-1]

    @pl.when(i == 0)
    def _prologue():
        for k in range(_K):
            pltpu.make_async_copy(x_ref.at[k], ibuf.at[k], isem.at[k]).start()

    pltpu.make_async_copy(x_ref.at[i], ibuf.at[slot], isem.at[slot]).wait()

    @pl.when(i >= _K)
    def _wait_out():
        pltpu.make_async_copy(obuf.at[slot], o_ref.at[i - _K], osem.at[slot]).wait()

    mid = mid_ref[i]
    pos = jax.lax.broadcasted_iota(jnp.int32, (1, hw), 1)
    obuf[slot] = jnp.where(pos == mid, 0.0, ibuf[slot])

    pltpu.make_async_copy(obuf.at[slot], o_ref.at[i], osem.at[slot]).start()

    @pl.when(i + _K < n)
    def _next_in():
        pltpu.make_async_copy(x_ref.at[i + _K], ibuf.at[slot], isem.at[slot]).start()

    @pl.when(i == n - 1)
    def _drain():
        for k in range(_K):
            j = i - (_K - 1) + k
            sl = jax.lax.rem(j, _K)
            pltpu.make_async_copy(obuf.at[sl], o_ref.at[j], osem.at[sl]).wait()


def kernel(fea, mask_id):
    b, c, h, w = fea.shape
    hw = h * w
    x = fea.reshape(b, c, hw)
    out = pl.pallas_call(
        _body,
        grid=(b,),
        in_specs=[
            pl.BlockSpec(memory_space=pl.ANY),
            pl.BlockSpec(memory_space=pltpu.SMEM),
        ],
        out_specs=pl.BlockSpec(memory_space=pl.ANY),
        out_shape=jax.ShapeDtypeStruct((b, c, hw), jnp.float32),
        scratch_shapes=[
            pltpu.VMEM((_K, c, hw), jnp.float32),
            pltpu.VMEM((_K, c, hw), jnp.float32),
            pltpu.SemaphoreType.DMA((_K,)),
            pltpu.SemaphoreType.DMA((_K,)),
        ],
    )(x, mask_id)
    return out.reshape(b, c, h, w)
